# baseline (speedup 1.0000x reference)
import functools

import jax
import jax.numpy as jnp
from jax import lax
from jax.experimental import pallas as pl
from jax.experimental.pallas import tpu as pltpu
from jax.experimental.pallas import tpu_sc as plsc

N = 10000
D = 128
G = 64
EPS = 1e-5

NC = 2
NS = 16
NW = NC * NS
K = 128
NP = 10240
PMAX = 56
BM = 2048
GRID = NP // BM
L = 16


def _sc_mesh():
    return plsc.VectorSubcoreMesh(core_axis_name="c", subcore_axis_name="s")


def _deg_body(dst_hbm, deg_hbm, dstv, hist):
    cid = lax.axis_index("c")
    sid = lax.axis_index("s")
    wid = sid * NC + cid
    pltpu.sync_copy(dst_hbm.at[wid], dstv)

    zero = jnp.zeros((L,), jnp.float32)

    def zloop(i, c):
        hist[pl.ds(i * L, L)] = zero
        return c

    lax.fori_loop(0, NP // L, zloop, 0)

    ones = jnp.full((L,), 1.0, jnp.float32)
    nvec = dstv.shape[0] // L

    def hloop(i, c):
        idx = dstv[pl.ds(i * L, L)]
        plsc.addupdate_scatter(hist, [idx], ones)
        return c

    lax.fori_loop(0, nvec, hloop, 0)
    pltpu.sync_copy(hist, deg_hbm.at[wid])


def _make_deg_kernel(cflat):
    return pl.kernel(
        _deg_body,
        out_type=jax.ShapeDtypeStruct((NW, NP), jnp.float32),
        mesh=_sc_mesh(),
        compiler_params=pltpu.CompilerParams(needs_layout_passes=False),
        scratch_types=[
            pltpu.VMEM((cflat,), jnp.int32),
            pltpu.VMEM((NP,), jnp.float32),
        ],
    )


def _msg_body(hws_hbm, src_hbm, dst_hbm, out_hbm, srcv, dstv, rows0, rows1,
              sem0, sem1, acc, n0=None, n1=None):
    cid = lax.axis_index("c")
    sid = lax.axis_index("s")

    zero = jnp.zeros((L,), jnp.float32)

    def zv(i, c):
        rows0[i // (D // L), pl.ds((i % (D // L)) * L, L)] = zero
        return c

    lax.fori_loop(0, K * (D // L), zv, 0)

    base = sid * (NP // NS)

    def zacc(t, c):
        pltpu.sync_copy(rows0, acc.at[pl.ds(base + t * K, K)])
        return c

    lax.fori_loop(0, (NP // NS) // K, zacc, 0)

    plsc.subcore_barrier()

    H = K // 2

    def gat(j, rows, sem):
        pltpu.async_copy(hws_hbm.at[srcv.at[j, pl.ds(0, H)]],
                         rows.at[pl.ds(0, H)], sem)
        pltpu.async_copy(hws_hbm.at[srcv.at[j, pl.ds(H, H)]],
                         rows.at[pl.ds(H, H)], sem)

    def gwait(j, rows, sem):
        pltpu.make_async_copy(hws_hbm.at[srcv.at[j, pl.ds(0, H)]],
                              rows.at[pl.ds(0, H)], sem).wait()
        pltpu.make_async_copy(hws_hbm.at[srcv.at[j, pl.ds(H, H)]],
                              rows.at[pl.ds(H, H)], sem).wait()

    def run_half(off, m):
        pltpu.sync_copy(src_hbm.at[sid, pl.ds(off, m)], srcv.at[pl.ds(0, m)])
        pltpu.sync_copy(dst_hbm.at[sid, pl.ds(off, m)], dstv.at[pl.ds(0, m)])
        gat(0, rows0, sem0)

        def pair(t, c):
            j = 2 * t
            gat(j + 1, rows1, sem1)
            gwait(j, rows0, sem0)
            pltpu.sync_copy(rows0, acc.at[dstv.at[j]], add=True)

            @pl.when(j + 2 < m)
            def _():
                gat(j + 2, rows0, sem0)

            gwait(j + 1, rows1, sem1)
            pltpu.sync_copy(rows1, acc.at[dstv.at[j + 1]], add=True)
            return c

        lax.fori_loop(0, m // 2, pair, 0)

    def parts(total):
        out, off = [], 0
        while off < total:
            m = min(PMAX, total - off)
            out.append((off, m))
            off += m
        return out

    @pl.when(cid == 0)
    def _():
        for off, m in parts(n0):
            run_half(off, m)

    @pl.when(cid == 1)
    def _():
        for off, m in parts(n1):
            run_half(n0 + off, m)

    plsc.subcore_barrier()
    pltpu.sync_copy(acc.at[pl.ds(base, NP // NS)],
                    out_hbm.at[cid, pl.ds(base, NP // NS)])


def _make_msg_kernel(n0, n1):
    nmax = PMAX
    return pl.kernel(
        functools.partial(_msg_body, n0=n0, n1=n1),
        out_type=jax.ShapeDtypeStruct((NC, NP, D), jnp.float32),
        mesh=_sc_mesh(),
        scratch_types=[
            pltpu.VMEM((nmax, K), jnp.int32),
            pltpu.VMEM((nmax, K), jnp.int32),
            pltpu.VMEM((K, D), jnp.float32),
            pltpu.VMEM((K, D), jnp.float32),
            pltpu.SemaphoreType.DMA,
            pltpu.SemaphoreType.DMA,
            pltpu.VMEM_SHARED((NP, D), jnp.float32),
        ],
    )


def _t1_body(dpart_ref, x_ref, w_ref, hws_ref, dinv_ref):
    ones = jnp.ones((NW, 1), jnp.float32)
    deg = lax.dot_general(dpart_ref[...], ones,
                          (((0,), (0,)), ((), ())),
                          preferred_element_type=jnp.float32)
    dinv = lax.rsqrt(deg + 1.0)
    dinv_ref[...] = dinv
    hw = jnp.dot(x_ref[...], w_ref[...], preferred_element_type=jnp.float32)
    hws_ref[...] = hw * dinv


def _t1_call(deg_part, x_p, W):
    return pl.pallas_call(
        _t1_body,
        grid=(GRID,),
        in_specs=[
            pl.BlockSpec((NW, BM), lambda i: (0, i)),
            pl.BlockSpec((BM, D), lambda i: (i, 0)),
            pl.BlockSpec((D, D), lambda i: (0, 0)),
        ],
        out_specs=[
            pl.BlockSpec((BM, D), lambda i: (i, 0)),
            pl.BlockSpec((BM, 1), lambda i: (i, 0)),
        ],
        out_shape=[
            jax.ShapeDtypeStruct((NP, D), jnp.float32),
            jax.ShapeDtypeStruct((NP, 1), jnp.float32),
        ],
    )(deg_part, x_p, W)


def _stats_body(msg_ref, hws_ref, dinv_ref, b_ref, m_ref, s_ref):
    i = pl.program_id(0)
    m = (msg_ref[0] + msg_ref[1] + hws_ref[...]) * dinv_ref[...] + b_ref[...]
    m_ref[...] = m
    rows = lax.broadcasted_iota(jnp.int32, (BM, 1), 0) + i * BM
    mask = (rows < N).astype(jnp.float32)
    mm = m * mask
    s = jnp.sum(mm, axis=0, keepdims=True)
    ss = jnp.sum(mm * mm, axis=0, keepdims=True)
    upd = jnp.concatenate([s, ss, jnp.zeros((6, D), jnp.float32)], axis=0)

    @pl.when(i == 0)
    def _():
        s_ref[...] = jnp.zeros((8, D), jnp.float32)

    s_ref[...] += upd


def _stats_call(msg, hws, dinv, b_row):
    return pl.pallas_call(
        _stats_body,
        grid=(GRID,),
        in_specs=[
            pl.BlockSpec((NC, BM, D), lambda i: (0, i, 0)),
            pl.BlockSpec((BM, D), lambda i: (i, 0)),
            pl.BlockSpec((BM, 1), lambda i: (i, 0)),
            pl.BlockSpec((1, D), lambda i: (0, 0)),
        ],
        out_specs=[
            pl.BlockSpec((BM, D), lambda i: (i, 0)),
            pl.BlockSpec((8, D), lambda i: (0, 0)),
        ],
        out_shape=[
            jax.ShapeDtypeStruct((NP, D), jnp.float32),
            jax.ShapeDtypeStruct((8, D), jnp.float32),
        ],
    )(msg, hws, dinv, b_row)


def _bn(m, s_ref, g_ref, be_ref):
    s = s_ref[0:1, :]
    ss = s_ref[1:2, :]
    mean = s * (1.0 / N)
    var = ss * (1.0 / N) - mean * mean
    inv = lax.rsqrt(var + EPS)
    return jnp.maximum((m - mean) * inv * g_ref[...] + be_ref[...], 0.0)


def _t2b_body(m_ref, s_ref, g_ref, be_ref, dinv_ref, w_ref, out_ref):
    h = _bn(m_ref[...], s_ref, g_ref, be_ref)
    out_ref[...] = jnp.dot(h, w_ref[...],
                           preferred_element_type=jnp.float32) * dinv_ref[...]


def _t2b_call(m1, stats, g_row, be_row, dinv, W):
    return pl.pallas_call(
        _t2b_body,
        grid=(GRID,),
        in_specs=[
            pl.BlockSpec((BM, D), lambda i: (i, 0)),
            pl.BlockSpec((8, D), lambda i: (0, 0)),
            pl.BlockSpec((1, D), lambda i: (0, 0)),
            pl.BlockSpec((1, D), lambda i: (0, 0)),
            pl.BlockSpec((BM, 1), lambda i: (i, 0)),
            pl.BlockSpec((D, D), lambda i: (0, 0)),
        ],
        out_specs=pl.BlockSpec((BM, D), lambda i: (i, 0)),
        out_shape=jax.ShapeDtypeStruct((NP, D), jnp.float32),
    )(m1, stats, g_row, be_row, dinv, W)


def _t3b_body(m_ref, s_ref, g_ref, be_ref, batch_ref, out_ref):
    i = pl.program_id(0)
    h = _bn(m_ref[...], s_ref, g_ref, be_ref)
    onehot = (batch_ref[...] ==
              lax.broadcasted_iota(jnp.int32, (1, G), 1)).astype(jnp.float32)
    p = lax.dot_general(onehot, h, (((0,), (0,)), ((), ())),
                        preferred_element_type=jnp.float32)

    @pl.when(i == 0)
    def _():
        out_ref[...] = jnp.zeros((G, D), jnp.float32)

    out_ref[...] += p


def _t3b_call(m2, stats, g_row, be_row, batch_p):
    return pl.pallas_call(
        _t3b_body,
        grid=(GRID,),
        in_specs=[
            pl.BlockSpec((BM, D), lambda i: (i, 0)),
            pl.BlockSpec((8, D), lambda i: (0, 0)),
            pl.BlockSpec((1, D), lambda i: (0, 0)),
            pl.BlockSpec((1, D), lambda i: (0, 0)),
            pl.BlockSpec((BM, 1), lambda i: (i, 0)),
        ],
        out_specs=pl.BlockSpec((G, D), lambda i: (0, 0)),
        out_shape=jax.ShapeDtypeStruct((G, D), jnp.float32),
    )(m2, stats, g_row, be_row, batch_p)


def kernel(x, edge_index, batch, W1, b1, gamma1, beta1, W2, b2, gamma2, beta2):
    E = edge_index.shape[1]
    ntot = 8 * (-(-E // (NS * K * 8)))
    n0 = 8 * round(ntot * 0.75 / 8)
    n1 = ntot - n0
    Ep = NS * ntot * K
    pad = Ep - E

    src = edge_index[0]
    dst = edge_index[1]
    src3 = jnp.concatenate(
        [src, jnp.zeros((pad,), jnp.int32)]).reshape(NS, ntot, K)
    dst_p = jnp.concatenate([dst, jnp.full((pad,), N, jnp.int32)])
    dst3 = dst_p.reshape(NS, ntot, K)
    dst_flat = dst_p.reshape(NW, (NS * ntot * K) // NW)

    x_p = jnp.concatenate([x, jnp.zeros((NP - N, D), jnp.float32)], axis=0)
    batch_p = jnp.concatenate(
        [batch, jnp.full((NP - N,), G, jnp.int32)]).reshape(NP, 1)
    b1r = b1.reshape(1, D)
    b2r = b2.reshape(1, D)
    g1r = gamma1.reshape(1, D)
    g2r = gamma2.reshape(1, D)
    be1r = beta1.reshape(1, D)
    be2r = beta2.reshape(1, D)

    deg_part = _make_deg_kernel((NS * ntot * K) // NW)(dst_flat)
    hws1, dinv = _t1_call(deg_part, x_p, W1)

    msg_kernel = _make_msg_kernel(n0, n1)
    msg1 = msg_kernel(hws1, src3, dst3)
    m1, stats1 = _stats_call(msg1, hws1, dinv, b1r)
    hws2 = _t2b_call(m1, stats1, g1r, be1r, dinv, W2)

    msg2 = msg_kernel(hws2, src3, dst3)
    m2, stats2 = _stats_call(msg2, hws2, dinv, b2r)
    return _t3b_call(m2, stats2, g2r, be2r, batch_p)


# speedup vs baseline: 1.0009x; 1.0009x over previous
import functools

import jax
import jax.numpy as jnp
from jax import lax
from jax.experimental import pallas as pl
from jax.experimental.pallas import tpu as pltpu
from jax.experimental.pallas import tpu_sc as plsc

N = 10000
D = 128
G = 64
EPS = 1e-5

NC = 2
NS = 16
NW = NC * NS
K = 128
NP = 10240
PMAX = 56
BM = 2048
GRID = NP // BM
L = 16


def _sc_mesh():
    return plsc.VectorSubcoreMesh(core_axis_name="c", subcore_axis_name="s")


def _deg_body(dst_hbm, deg_hbm, dstv, hist):
    cid = lax.axis_index("c")
    sid = lax.axis_index("s")
    wid = sid * NC + cid
    pltpu.sync_copy(dst_hbm.at[wid], dstv)

    zero = jnp.zeros((L,), jnp.float32)

    def zloop(i, c):
        hist[pl.ds(i * L, L)] = zero
        return c

    lax.fori_loop(0, NP // L, zloop, 0)

    ones = jnp.full((L,), 1.0, jnp.float32)
    nvec = dstv.shape[0] // L

    def hloop(i, c):
        idx = dstv[pl.ds(i * L, L)]
        plsc.addupdate_scatter(hist, [idx], ones)
        return c

    lax.fori_loop(0, nvec, hloop, 0)
    pltpu.sync_copy(hist, deg_hbm.at[wid])


def _make_deg_kernel(cflat):
    return pl.kernel(
        _deg_body,
        out_type=jax.ShapeDtypeStruct((NW, NP), jnp.float32),
        mesh=_sc_mesh(),
        compiler_params=pltpu.CompilerParams(needs_layout_passes=False),
        scratch_types=[
            pltpu.VMEM((cflat,), jnp.int32),
            pltpu.VMEM((NP,), jnp.float32),
        ],
    )


def _msg_body(hws_hbm, src_hbm, dst_hbm, out_hbm, srcv, dstv, rows0, rows1,
              sem0, sem1, acc, n0=None, n1=None):
    cid = lax.axis_index("c")
    sid = lax.axis_index("s")

    zero = jnp.zeros((L,), jnp.float32)

    def zv(i, c):
        rows0[i // (D // L), pl.ds((i % (D // L)) * L, L)] = zero
        return c

    lax.fori_loop(0, K * (D // L), zv, 0)

    base = sid * (NP // NS)

    def zacc(t, c):
        pltpu.sync_copy(rows0, acc.at[pl.ds(base + t * K, K)])
        return c

    lax.fori_loop(0, (NP // NS) // K, zacc, 0)

    plsc.subcore_barrier()

    def run_half(off, m):
        pltpu.sync_copy(src_hbm.at[sid, pl.ds(off, m)], srcv.at[pl.ds(0, m)])
        pltpu.sync_copy(dst_hbm.at[sid, pl.ds(off, m)], dstv.at[pl.ds(0, m)])
        pltpu.async_copy(hws_hbm.at[srcv.at[0]], rows0, sem0)

        def pair(t, c):
            j = 2 * t
            pltpu.async_copy(hws_hbm.at[srcv.at[j + 1]], rows1, sem1)
            pltpu.make_async_copy(hws_hbm.at[srcv.at[j]], rows0, sem0).wait()
            pltpu.sync_copy(rows0, acc.at[dstv.at[j]], add=True)

            @pl.when(j + 2 < m)
            def _():
                pltpu.async_copy(hws_hbm.at[srcv.at[j + 2]], rows0, sem0)

            pltpu.make_async_copy(hws_hbm.at[srcv.at[j + 1]], rows1,
                                  sem1).wait()
            pltpu.sync_copy(rows1, acc.at[dstv.at[j + 1]], add=True)
            return c

        lax.fori_loop(0, m // 2, pair, 0)

    def parts(total):
        out, off = [], 0
        while off < total:
            m = min(PMAX, total - off)
            out.append((off, m))
            off += m
        return out

    @pl.when(cid == 0)
    def _():
        for off, m in parts(n0):
            run_half(off, m)

    @pl.when(cid == 1)
    def _():
        for off, m in parts(n1):
            run_half(n0 + off, m)

    plsc.subcore_barrier()
    pltpu.sync_copy(acc.at[pl.ds(base, NP // NS)],
                    out_hbm.at[cid, pl.ds(base, NP // NS)])


def _make_msg_kernel(n0, n1):
    nmax = PMAX
    return pl.kernel(
        functools.partial(_msg_body, n0=n0, n1=n1),
        out_type=jax.ShapeDtypeStruct((NC, NP, D), jnp.float32),
        mesh=_sc_mesh(),
        scratch_types=[
            pltpu.VMEM((nmax, K), jnp.int32),
            pltpu.VMEM((nmax, K), jnp.int32),
            pltpu.VMEM((K, D), jnp.float32),
            pltpu.VMEM((K, D), jnp.float32),
            pltpu.SemaphoreType.DMA,
            pltpu.SemaphoreType.DMA,
            pltpu.VMEM_SHARED((NP, D), jnp.float32),
        ],
    )


def _t1_body(dpart_ref, x_ref, w_ref, hws_ref, dinv_ref):
    ones = jnp.ones((NW, 1), jnp.float32)
    deg = lax.dot_general(dpart_ref[...], ones,
                          (((0,), (0,)), ((), ())),
                          preferred_element_type=jnp.float32)
    dinv = lax.rsqrt(deg + 1.0)
    dinv_ref[...] = dinv
    hw = jnp.dot(x_ref[...], w_ref[...], preferred_element_type=jnp.float32)
    hws_ref[...] = hw * dinv


def _t1_call(deg_part, x_p, W):
    return pl.pallas_call(
        _t1_body,
        grid=(GRID,),
        in_specs=[
            pl.BlockSpec((NW, BM), lambda i: (0, i)),
            pl.BlockSpec((BM, D), lambda i: (i, 0)),
            pl.BlockSpec((D, D), lambda i: (0, 0)),
        ],
        out_specs=[
            pl.BlockSpec((BM, D), lambda i: (i, 0)),
            pl.BlockSpec((BM, 1), lambda i: (i, 0)),
        ],
        out_shape=[
            jax.ShapeDtypeStruct((NP, D), jnp.float32),
            jax.ShapeDtypeStruct((NP, 1), jnp.float32),
        ],
    )(deg_part, x_p, W)


def _stats_body(msg_ref, hws_ref, dinv_ref, b_ref, m_ref, s_ref):
    i = pl.program_id(0)
    m = (msg_ref[0] + msg_ref[1] + hws_ref[...]) * dinv_ref[...] + b_ref[...]
    m_ref[...] = m
    rows = lax.broadcasted_iota(jnp.int32, (BM, 1), 0) + i * BM
    mask = (rows < N).astype(jnp.float32)
    mm = m * mask
    s = jnp.sum(mm, axis=0, keepdims=True)
    ss = jnp.sum(mm * mm, axis=0, keepdims=True)
    upd = jnp.concatenate([s, ss, jnp.zeros((6, D), jnp.float32)], axis=0)

    @pl.when(i == 0)
    def _():
        s_ref[...] = jnp.zeros((8, D), jnp.float32)

    s_ref[...] += upd


def _stats_call(msg, hws, dinv, b_row):
    return pl.pallas_call(
        _stats_body,
        grid=(GRID,),
        in_specs=[
            pl.BlockSpec((NC, BM, D), lambda i: (0, i, 0)),
            pl.BlockSpec((BM, D), lambda i: (i, 0)),
            pl.BlockSpec((BM, 1), lambda i: (i, 0)),
            pl.BlockSpec((1, D), lambda i: (0, 0)),
        ],
        out_specs=[
            pl.BlockSpec((BM, D), lambda i: (i, 0)),
            pl.BlockSpec((8, D), lambda i: (0, 0)),
        ],
        out_shape=[
            jax.ShapeDtypeStruct((NP, D), jnp.float32),
            jax.ShapeDtypeStruct((8, D), jnp.float32),
        ],
    )(msg, hws, dinv, b_row)


def _bn(m, s_ref, g_ref, be_ref):
    s = s_ref[0:1, :]
    ss = s_ref[1:2, :]
    mean = s * (1.0 / N)
    var = ss * (1.0 / N) - mean * mean
    inv = lax.rsqrt(var + EPS)
    return jnp.maximum((m - mean) * inv * g_ref[...] + be_ref[...], 0.0)


def _t2b_body(m_ref, s_ref, g_ref, be_ref, dinv_ref, w_ref, out_ref):
    h = _bn(m_ref[...], s_ref, g_ref, be_ref)
    out_ref[...] = jnp.dot(h, w_ref[...],
                           preferred_element_type=jnp.float32) * dinv_ref[...]


def _t2b_call(m1, stats, g_row, be_row, dinv, W):
    return pl.pallas_call(
        _t2b_body,
        grid=(GRID,),
        in_specs=[
            pl.BlockSpec((BM, D), lambda i: (i, 0)),
            pl.BlockSpec((8, D), lambda i: (0, 0)),
            pl.BlockSpec((1, D), lambda i: (0, 0)),
            pl.BlockSpec((1, D), lambda i: (0, 0)),
            pl.BlockSpec((BM, 1), lambda i: (i, 0)),
            pl.BlockSpec((D, D), lambda i: (0, 0)),
        ],
        out_specs=pl.BlockSpec((BM, D), lambda i: (i, 0)),
        out_shape=jax.ShapeDtypeStruct((NP, D), jnp.float32),
    )(m1, stats, g_row, be_row, dinv, W)


def _t3b_body(m_ref, s_ref, g_ref, be_ref, batch_ref, out_ref):
    i = pl.program_id(0)
    h = _bn(m_ref[...], s_ref, g_ref, be_ref)
    onehot = (batch_ref[...] ==
              lax.broadcasted_iota(jnp.int32, (1, G), 1)).astype(jnp.float32)
    p = lax.dot_general(onehot, h, (((0,), (0,)), ((), ())),
                        preferred_element_type=jnp.float32)

    @pl.when(i == 0)
    def _():
        out_ref[...] = jnp.zeros((G, D), jnp.float32)

    out_ref[...] += p


def _t3b_call(m2, stats, g_row, be_row, batch_p):
    return pl.pallas_call(
        _t3b_body,
        grid=(GRID,),
        in_specs=[
            pl.BlockSpec((BM, D), lambda i: (i, 0)),
            pl.BlockSpec((8, D), lambda i: (0, 0)),
            pl.BlockSpec((1, D), lambda i: (0, 0)),
            pl.BlockSpec((1, D), lambda i: (0, 0)),
            pl.BlockSpec((BM, 1), lambda i: (i, 0)),
        ],
        out_specs=pl.BlockSpec((G, D), lambda i: (0, 0)),
        out_shape=jax.ShapeDtypeStruct((G, D), jnp.float32),
    )(m2, stats, g_row, be_row, batch_p)


def kernel(x, edge_index, batch, W1, b1, gamma1, beta1, W2, b2, gamma2, beta2):
    E = edge_index.shape[1]
    ntot = 8 * (-(-E // (NS * K * 8)))
    n0 = 8 * round(ntot * 0.75 / 8)
    n1 = ntot - n0
    Ep = NS * ntot * K
    pad = Ep - E

    src = edge_index[0]
    dst = edge_index[1]
    src3 = jnp.concatenate(
        [src, jnp.zeros((pad,), jnp.int32)]).reshape(NS, ntot, K)
    dst_p = jnp.concatenate([dst, jnp.full((pad,), N, jnp.int32)])
    dst3 = dst_p.reshape(NS, ntot, K)
    dst_flat = dst_p.reshape(NW, (NS * ntot * K) // NW)

    x_p = jnp.concatenate([x, jnp.zeros((NP - N, D), jnp.float32)], axis=0)
    batch_p = jnp.concatenate(
        [batch, jnp.full((NP - N,), G, jnp.int32)]).reshape(NP, 1)
    b1r = b1.reshape(1, D)
    b2r = b2.reshape(1, D)
    g1r = gamma1.reshape(1, D)
    g2r = gamma2.reshape(1, D)
    be1r = beta1.reshape(1, D)
    be2r = beta2.reshape(1, D)

    deg_part = _make_deg_kernel((NS * ntot * K) // NW)(dst_flat)
    hws1, dinv = _t1_call(deg_part, x_p, W1)

    msg_kernel = _make_msg_kernel(n0, n1)
    msg1 = msg_kernel(hws1, src3, dst3)
    m1, stats1 = _stats_call(msg1, hws1, dinv, b1r)
    hws2 = _t2b_call(m1, stats1, g1r, be1r, dinv, W2)

    msg2 = msg_kernel(hws2, src3, dst3)
    m2, stats2 = _stats_call(msg2, hws2, dinv, b2r)
    return _t3b_call(m2, stats2, g2r, be2r, batch_p)
